# Initial kernel scaffold; baseline (speedup 1.0000x reference)
import jax
import jax.numpy as jnp
from jax.experimental import pallas as pl


def kernel(points):
    raise NotImplementedError("write your pallas kernel here")


# speedup vs baseline: 5.0838x; 5.0838x over previous
import jax
import jax.numpy as jnp
from jax import lax
from jax.experimental import pallas as pl
from jax.experimental.pallas import tpu as pltpu
from jax.experimental.pallas import tpu_sc as plsc
import functools

_B = 16
_N = 8192
_S = 2048
_L = 16
_CHUNKS = _N // _L


def _fps_body(pts_hbm, idx_hbm, xyz_hbm, x_ref, y_ref, z_ref, dist_ref,
              idx_ref, xyz_ref):
    c = lax.axis_index("c")
    s = lax.axis_index("s")

    @pl.when(c == 0)
    def _():
        b = s
        base = b * 3 * _N
        pltpu.sync_copy(pts_hbm.at[pl.ds(base, _N)], x_ref)
        pltpu.sync_copy(pts_hbm.at[pl.ds(base + _N, _N)], y_ref)
        pltpu.sync_copy(pts_hbm.at[pl.ds(base + 2 * _N, _N)], z_ref)

        lanes = lax.iota(jnp.int32, _L)
        lane0 = lanes == 0
        big = jnp.full((_L,), 1e10, jnp.float32)

        def init_dist(j, _):
            dist_ref[pl.ds(j * _L, _L)] = big
            return 0

        lax.fori_loop(0, _CHUNKS, init_dist, 0)

        def step(t, fv):
            cx = plsc.load_gather(x_ref, [fv])
            cy = plsc.load_gather(y_ref, [fv])
            cz = plsc.load_gather(z_ref, [fv])
            tv = lax.broadcast_in_dim(t, (_L,), ())
            plsc.store_scatter(idx_ref, [tv], fv, mask=lane0)
            t3 = tv * 3
            plsc.store_scatter(xyz_ref, [t3], cx, mask=lane0)
            plsc.store_scatter(xyz_ref, [t3 + 1], cy, mask=lane0)
            plsc.store_scatter(xyz_ref, [t3 + 2], cz, mask=lane0)

            def chunk(j, carry):
                best_v, best_i = carry
                sl = pl.ds(j * _L, _L)
                dx = x_ref[sl] - cx
                dy = y_ref[sl] - cy
                dz = z_ref[sl] - cz
                d = (dx * dx + dz * dz) + dy * dy
                nd = jnp.minimum(dist_ref[sl], d)
                dist_ref[sl] = nd
                gidx = lanes + j * _L
                upd = nd > best_v
                best_v = jnp.where(upd, nd, best_v)
                best_i = jnp.where(upd, gidx, best_i)
                return best_v, best_i

            best_v, best_i = lax.fori_loop(
                0, _CHUNKS, chunk,
                (jnp.full((_L,), -1.0, jnp.float32), jnp.zeros((_L,), jnp.int32)))
            m = jnp.max(best_v)
            cand = jnp.where(best_v == m, best_i,
                             jnp.full((_L,), 2147483647, jnp.int32))
            nf = jnp.min(cand)
            return lax.broadcast_in_dim(nf, (_L,), ())

        lax.fori_loop(0, _S, step, jnp.zeros((_L,), jnp.int32))

        pltpu.sync_copy(idx_ref, idx_hbm.at[pl.ds(b * _S, _S)])
        pltpu.sync_copy(xyz_ref, xyz_hbm.at[pl.ds(b * _S * 3, _S * 3)])


@jax.jit
def _fps_call(pts):
    mesh = plsc.VectorSubcoreMesh(core_axis_name="c", subcore_axis_name="s")
    k = functools.partial(
        pl.kernel,
        mesh=mesh,
        compiler_params=pltpu.CompilerParams(needs_layout_passes=False),
        out_type=(
            jax.ShapeDtypeStruct((_B * _S,), jnp.int32),
            jax.ShapeDtypeStruct((_B * _S * 3,), jnp.float32),
        ),
        scratch_types=[
            pltpu.VMEM((_N,), jnp.float32),
            pltpu.VMEM((_N,), jnp.float32),
            pltpu.VMEM((_N,), jnp.float32),
            pltpu.VMEM((_N,), jnp.float32),
            pltpu.VMEM((_S,), jnp.int32),
            pltpu.VMEM((_S * 3,), jnp.float32),
        ],
    )(_fps_body)
    return k(pts)


def kernel(points):
    pts = jnp.transpose(points, (0, 2, 1)).reshape(-1)
    idx, xyz = _fps_call(pts)
    return idx.reshape(_B, _S), xyz.reshape(_B, _S, 3)


# speedup vs baseline: 17.4763x; 3.4377x over previous
import jax
import jax.numpy as jnp
from jax import lax
from jax.experimental import pallas as pl
from jax.experimental.pallas import tpu as pltpu
from jax.experimental.pallas import tpu_sc as plsc
import functools

_B = 16
_N = 8192
_S = 2048
_L = 16
_CHUNKS = _N // _L


def _fps_body(pts_hbm, idx_hbm, xyz_hbm, x_ref, y_ref, z_ref, dist_ref,
              idx_ref, xyz_ref):
    c = lax.axis_index("c")
    s = lax.axis_index("s")

    @pl.when(c == 0)
    def _():
        b = s
        base = b * 3 * _N
        pltpu.sync_copy(pts_hbm.at[pl.ds(base, _N)], x_ref)
        pltpu.sync_copy(pts_hbm.at[pl.ds(base + _N, _N)], y_ref)
        pltpu.sync_copy(pts_hbm.at[pl.ds(base + 2 * _N, _N)], z_ref)

        lanes = lax.iota(jnp.int32, _L)
        lane0 = lanes == 0
        big = jnp.full((_L,), 1e10, jnp.float32)

        @plsc.parallel_loop(0, _CHUNKS, unroll=8)
        def _init(j):
            dist_ref[pl.ds(j * _L, _L)] = big

        def step(t, fv):
            cx = plsc.load_gather(x_ref, [fv])
            cy = plsc.load_gather(y_ref, [fv])
            cz = plsc.load_gather(z_ref, [fv])
            tv = lax.broadcast_in_dim(t, (_L,), ())
            plsc.store_scatter(idx_ref, [tv], fv, mask=lane0)
            t3 = tv * 3
            plsc.store_scatter(xyz_ref, [t3], cx, mask=lane0)
            plsc.store_scatter(xyz_ref, [t3 + 1], cy, mask=lane0)
            plsc.store_scatter(xyz_ref, [t3 + 2], cz, mask=lane0)

            init = (jnp.full((_L,), -1.0, jnp.float32),
                    jnp.zeros((_L,), jnp.int32))

            @plsc.parallel_loop(0, _CHUNKS, unroll=4, carry=init)
            def chunk(j, carry):
                best_v, best_i = carry
                sl = pl.ds(j * _L, _L)
                dx = x_ref[sl] - cx
                dy = y_ref[sl] - cy
                dz = z_ref[sl] - cz
                d = (dx * dx + dz * dz) + dy * dy
                nd = jnp.minimum(dist_ref[sl], d)
                dist_ref[sl] = nd
                gidx = lanes + j * _L
                upd = nd > best_v
                best_v = jnp.where(upd, nd, best_v)
                best_i = jnp.where(upd, gidx, best_i)
                return best_v, best_i

            best_v, best_i = chunk
            m = jnp.max(best_v)
            cand = jnp.where(best_v == m, best_i,
                             jnp.full((_L,), 2147483647, jnp.int32))
            nf = jnp.min(cand)
            return lax.broadcast_in_dim(nf, (_L,), ())

        lax.fori_loop(0, _S, step, jnp.zeros((_L,), jnp.int32))

        pltpu.sync_copy(idx_ref, idx_hbm.at[pl.ds(b * _S, _S)])
        pltpu.sync_copy(xyz_ref, xyz_hbm.at[pl.ds(b * _S * 3, _S * 3)])


@jax.jit
def _fps_call(pts):
    mesh = plsc.VectorSubcoreMesh(core_axis_name="c", subcore_axis_name="s")
    k = functools.partial(
        pl.kernel,
        mesh=mesh,
        compiler_params=pltpu.CompilerParams(needs_layout_passes=False),
        out_type=(
            jax.ShapeDtypeStruct((_B * _S,), jnp.int32),
            jax.ShapeDtypeStruct((_B * _S * 3,), jnp.float32),
        ),
        scratch_types=[
            pltpu.VMEM((_N,), jnp.float32),
            pltpu.VMEM((_N,), jnp.float32),
            pltpu.VMEM((_N,), jnp.float32),
            pltpu.VMEM((_N,), jnp.float32),
            pltpu.VMEM((_S,), jnp.int32),
            pltpu.VMEM((_S * 3,), jnp.float32),
        ],
    )(_fps_body)
    return k(pts)


def kernel(points):
    pts = jnp.transpose(points, (0, 2, 1)).reshape(-1)
    idx, xyz = _fps_call(pts)
    return idx.reshape(_B, _S), xyz.reshape(_B, _S, 3)


# speedup vs baseline: 17.9391x; 1.0265x over previous
import jax
import jax.numpy as jnp
from jax import lax
from jax.experimental import pallas as pl
from jax.experimental.pallas import tpu as pltpu
from jax.experimental.pallas import tpu_sc as plsc
import functools

_B = 16
_N = 8192
_S = 2048
_L = 16
_CHUNKS = _N // _L


def _fps_body(pts_hbm, idx_hbm, xyz_hbm, x_ref, y_ref, z_ref, dist_ref,
              idx_ref, xyz_ref):
    c = lax.axis_index("c")
    s = lax.axis_index("s")

    @pl.when(c == 0)
    def _():
        b = s
        base = b * 3 * _N
        pltpu.sync_copy(pts_hbm.at[pl.ds(base, _N)], x_ref)
        pltpu.sync_copy(pts_hbm.at[pl.ds(base + _N, _N)], y_ref)
        pltpu.sync_copy(pts_hbm.at[pl.ds(base + 2 * _N, _N)], z_ref)

        lanes = lax.iota(jnp.int32, _L)
        lane0 = lanes == 0
        big = jnp.full((_L,), 1e10, jnp.float32)

        @plsc.parallel_loop(0, _CHUNKS, unroll=8)
        def _init(j):
            dist_ref[pl.ds(j * _L, _L)] = big

        def step(t, fv):
            cx = plsc.load_gather(x_ref, [fv])
            cy = plsc.load_gather(y_ref, [fv])
            cz = plsc.load_gather(z_ref, [fv])
            tv = lax.broadcast_in_dim(t, (_L,), ())
            plsc.store_scatter(idx_ref, [tv], fv, mask=lane0)
            t3 = tv * 3
            plsc.store_scatter(xyz_ref, [t3], cx, mask=lane0)
            plsc.store_scatter(xyz_ref, [t3 + 1], cy, mask=lane0)
            plsc.store_scatter(xyz_ref, [t3 + 2], cz, mask=lane0)

            init = (jnp.full((_L,), -1.0, jnp.float32),
                    jnp.zeros((_L,), jnp.int32))

            @plsc.parallel_loop(0, _CHUNKS, unroll=8, carry=init)
            def chunk(j, carry):
                best_v, best_i = carry
                sl = pl.ds(j * _L, _L)
                dx = x_ref[sl] - cx
                dy = y_ref[sl] - cy
                dz = z_ref[sl] - cz
                d = (dx * dx + dz * dz) + dy * dy
                nd = jnp.minimum(dist_ref[sl], d)
                dist_ref[sl] = nd
                gidx = lanes + j * _L
                upd = nd > best_v
                best_v = jnp.where(upd, nd, best_v)
                best_i = jnp.where(upd, gidx, best_i)
                return best_v, best_i

            best_v, best_i = chunk
            m = jnp.max(best_v)
            cand = jnp.where(best_v == m, best_i,
                             jnp.full((_L,), 2147483647, jnp.int32))
            nf = jnp.min(cand)
            return lax.broadcast_in_dim(nf, (_L,), ())

        lax.fori_loop(0, _S, step, jnp.zeros((_L,), jnp.int32))

        pltpu.sync_copy(idx_ref, idx_hbm.at[pl.ds(b * _S, _S)])
        pltpu.sync_copy(xyz_ref, xyz_hbm.at[pl.ds(b * _S * 3, _S * 3)])


@jax.jit
def _fps_call(pts):
    mesh = plsc.VectorSubcoreMesh(core_axis_name="c", subcore_axis_name="s")
    k = functools.partial(
        pl.kernel,
        mesh=mesh,
        compiler_params=pltpu.CompilerParams(needs_layout_passes=False),
        out_type=(
            jax.ShapeDtypeStruct((_B * _S,), jnp.int32),
            jax.ShapeDtypeStruct((_B * _S * 3,), jnp.float32),
        ),
        scratch_types=[
            pltpu.VMEM((_N,), jnp.float32),
            pltpu.VMEM((_N,), jnp.float32),
            pltpu.VMEM((_N,), jnp.float32),
            pltpu.VMEM((_N,), jnp.float32),
            pltpu.VMEM((_S,), jnp.int32),
            pltpu.VMEM((_S * 3,), jnp.float32),
        ],
    )(_fps_body)
    return k(pts)


def kernel(points):
    pts = jnp.transpose(points, (0, 2, 1)).reshape(-1)
    idx, xyz = _fps_call(pts)
    return idx.reshape(_B, _S), xyz.reshape(_B, _S, 3)
